# baseline (speedup 1.0000x reference)
import jax
import jax.numpy as jnp
from jax import lax
from jax.experimental import pallas as pl
from jax.experimental.pallas import tpu as pltpu
from jax.experimental.pallas import tpu_sc as plsc

_C = 8
_THR = 0.8
_L = 16
_NK = 4
_NS = 4


def _accum_stream(k, x_ref, lab_ref, out_ref, w, b, sum_ref, cnt_ref):
    x = x_ref[0, 0]
    nch = x.shape[0]
    lab = lab_ref[0, 0]
    cid = jax.lax.broadcasted_iota(jnp.int32, (nch, _C), 1)
    onehot = (lab == cid).astype(jnp.float32)
    psum = jax.lax.dot_general(
        onehot, x, (((0,), (0,)), ((), ())),
        preferred_element_type=jnp.float32)
    ones = jnp.ones((nch, _C), jnp.float32)
    pcnt = jax.lax.dot_general(
        onehot, ones, (((0,), (0,)), ((), ())),
        preferred_element_type=jnp.float32)

    @pl.when(k == 0)
    def _init():
        sum_ref[...] = psum
        cnt_ref[...] = pcnt

    @pl.when(k != 0)
    def _acc():
        sum_ref[...] += psum
        cnt_ref[...] += pcnt

    @pl.when(k == _NK - 1)
    def _fin():
        cnt = jnp.maximum(cnt_ref[:, 0:1], 1.0)
        feats = sum_ref[...] / cnt
        logits = jax.lax.dot_general(
            feats, w, (((1,), (0,)), ((), ())),
            preferred_element_type=jnp.float32) + b
        out_ref[0] = jnp.pad(logits, ((0, 0), (0, 128 - logits.shape[1])))


def _tc_body(w_ref, b_ref, *refs):
    k = pl.program_id(1)
    w = w_ref[...]
    b = b_ref[...]
    x_refs = refs[0:_NS]
    lab_refs = refs[_NS:2 * _NS]
    out_refs = refs[2 * _NS:3 * _NS]
    sum_refs = refs[3 * _NS:4 * _NS]
    cnt_refs = refs[4 * _NS:5 * _NS]
    for s in range(_NS):
        _accum_stream(k, x_refs[s], lab_refs[s], out_refs[s], w, b,
                      sum_refs[s], cnt_refs[s])


def _sc_body(*refs):
    seg_hbm = refs[0:_NS]
    feats_hbm = refs[_NS]
    scores_hbm = refs[_NS + 1]
    rowv = refs[_NS + 2]
    outv = refs[_NS + 3]
    cidx = lax.axis_index("c")
    sidx = lax.axis_index("s")

    @pl.when(sidx < 4)
    def _leader():
        bag = cidx * 4 + sidx
        stream = bag // 2
        for s in range(_NS):
            @pl.when(stream == s)
            def _cp(s=s):
                pltpu.sync_copy(
                    seg_hbm[s].at[pl.ds((bag % 2) * _C * 128, _C * 128)],
                    rowv)

        lane = lax.iota(jnp.int32, _L)
        base = lane * 128
        l0 = plsc.load_gather(rowv, [base])
        l1 = plsc.load_gather(rowv, [base + 1])

        m = jnp.maximum(l0, l1)
        e0 = jnp.exp(l0 - m)
        e1 = jnp.exp(l1 - m)
        sc = e1 / (e0 + e1)
        valid = lane < _C
        scm = jnp.where(valid, sc, -1.0)
        scp = jnp.where(valid, sc, 2.0)
        mx = jnp.max(scm)
        mn = jnp.min(scp)
        idx_max = plsc.all_reduce_ffs(scm == mx)
        idx_min = plsc.all_reduce_ffs(scp == mn)
        sel = jnp.where(mx < _THR, idx_min, idx_max)
        neg = jnp.float32(-3.0e38)
        l0s = jnp.max(jnp.where(lane == sel, l0, neg))
        l1s = jnp.max(jnp.where(lane == sel, l1, neg))
        outv[...] = jnp.where(lane == 0, l0s,
                              jnp.where(lane == 1, l1s, 0.0))
        pltpu.sync_copy(outv, feats_hbm.at[pl.ds(bag * _L, _L)])
        outv[...] = jnp.where(valid, sc, 0.0)
        pltpu.sync_copy(outv, scores_hbm.at[pl.ds(bag * _L, _L)])


def kernel(inst_feat, cluster_labels, W_head, b_head):
    B, N, D = inst_feat.shape
    ncls = W_head.shape[1]
    nch = N // _NK
    bps = B // _NS

    xs = [inst_feat[s * bps:(s + 1) * bps].reshape(bps, _NK, nch, D)
          for s in range(_NS)]
    labs = [cluster_labels[s * bps:(s + 1) * bps].reshape(bps, _NK, nch, 1)
            for s in range(_NS)]

    x_spec = pl.BlockSpec((1, 1, nch, D), lambda b, k: (b, k, 0, 0))
    lab_spec = pl.BlockSpec((1, 1, nch, 1), lambda b, k: (b, k, 0, 0))
    out_spec = pl.BlockSpec((1, _C, 128), lambda b, k: (b, 0, 0))

    segs = pl.pallas_call(
        _tc_body,
        grid=(bps, _NK),
        in_specs=[
            pl.BlockSpec((D, ncls), lambda b, k: (0, 0)),
            pl.BlockSpec((1, ncls), lambda b, k: (0, 0)),
        ] + [x_spec] * _NS + [lab_spec] * _NS,
        out_specs=[out_spec] * _NS,
        out_shape=[jax.ShapeDtypeStruct((bps, _C, 128), jnp.float32)] * _NS,
        scratch_shapes=(
            [pltpu.VMEM((_C, D), jnp.float32)] * _NS
            + [pltpu.VMEM((_C, _C), jnp.float32)] * _NS),
    )(W_head, b_head.reshape(1, ncls), *xs, *labs)

    mesh = plsc.VectorSubcoreMesh(core_axis_name="c", subcore_axis_name="s")
    sc_call = pl.kernel(
        _sc_body,
        out_type=(
            jax.ShapeDtypeStruct((B * _L,), jnp.float32),
            jax.ShapeDtypeStruct((B * _L,), jnp.float32),
        ),
        mesh=mesh,
        compiler_params=pltpu.CompilerParams(needs_layout_passes=False),
        scratch_types=[
            pltpu.VMEM((_C * 128,), jnp.float32),
            pltpu.VMEM((_L,), jnp.float32),
        ],
    )
    featsp, scoresp = sc_call(*[s.reshape(-1) for s in segs])
    feats = featsp.reshape(B, _L)[:, :ncls]
    scores = scoresp.reshape(B, _L)[:, :_C]
    return feats, scores


# speedup vs baseline: 1.9489x; 1.9489x over previous
import jax
import jax.numpy as jnp
from jax import lax
from jax.experimental import pallas as pl
from jax.experimental.pallas import tpu as pltpu
from jax.experimental.pallas import tpu_sc as plsc

_C = 8
_THR = 0.8
_L = 16
_NK = 4
_NB = 4


def _tc_body(w_ref, b_ref, x_hbm, lab_ref, out_ref,
             buf_ref, sum_ref, cnt_ref, sems):
    bi = pl.program_id(0)
    k = pl.program_id(1)
    i = bi * _NK + k
    nsteps = pl.num_programs(0) * _NK
    nch = buf_ref.shape[1]

    def _start(chunk, slot):
        bag = chunk // _NK
        kk = chunk % _NK
        pltpu.make_async_copy(
            x_hbm.at[bag, pl.ds(kk * nch, nch), :],
            buf_ref.at[slot],
            sems.at[slot],
        ).start()

    @pl.when(i == 0)
    def _prime():
        for j in range(_NB):
            _start(j, j)

    slot = lax.rem(i, _NB)
    pltpu.make_async_copy(
        x_hbm.at[bi, pl.ds(k * nch, nch), :],
        buf_ref.at[slot],
        sems.at[slot],
    ).wait()

    x = buf_ref[slot]
    lab = lab_ref[0]
    cid = jax.lax.broadcasted_iota(jnp.int32, (nch, _C), 1)
    onehot = (lab == cid).astype(jnp.float32)
    psum = jax.lax.dot_general(
        onehot, x, (((0,), (0,)), ((), ())),
        preferred_element_type=jnp.float32)
    ones = jnp.ones((nch, _C), jnp.float32)
    pcnt = jax.lax.dot_general(
        onehot, ones, (((0,), (0,)), ((), ())),
        preferred_element_type=jnp.float32)

    @pl.when(k == 0)
    def _init():
        sum_ref[...] = psum
        cnt_ref[...] = pcnt

    @pl.when(k != 0)
    def _acc():
        sum_ref[...] += psum
        cnt_ref[...] += pcnt

    nxt = i + _NB

    @pl.when(nxt < nsteps)
    def _refill():
        bag2 = lax.div(nxt, _NK)
        k2 = lax.rem(nxt, _NK)
        pltpu.make_async_copy(
            x_hbm.at[bag2, pl.ds(k2 * nch, nch), :],
            buf_ref.at[slot],
            sems.at[slot],
        ).start()

    @pl.when(k == _NK - 1)
    def _fin():
        cnt = jnp.maximum(cnt_ref[:, 0:1], 1.0)
        feats = sum_ref[...] / cnt
        logits = jax.lax.dot_general(
            feats, w_ref[...], (((1,), (0,)), ((), ())),
            preferred_element_type=jnp.float32)
        logits = logits + b_ref[...]
        out_ref[0] = jnp.pad(logits, ((0, 0), (0, 128 - logits.shape[1])))


def _sc_body(seg_hbm, feats_hbm, scores_hbm, rowv, outv):
    cidx = lax.axis_index("c")
    sidx = lax.axis_index("s")

    @pl.when(sidx < 4)
    def _leader():
        bag = cidx * 4 + sidx
        pltpu.sync_copy(seg_hbm.at[pl.ds(bag * _C * 128, _C * 128)], rowv)

        lane = lax.iota(jnp.int32, _L)
        base = lane * 128
        l0 = plsc.load_gather(rowv, [base])
        l1 = plsc.load_gather(rowv, [base + 1])

        m = jnp.maximum(l0, l1)
        e0 = jnp.exp(l0 - m)
        e1 = jnp.exp(l1 - m)
        sc = e1 / (e0 + e1)
        valid = lane < _C
        scm = jnp.where(valid, sc, -1.0)
        scp = jnp.where(valid, sc, 2.0)
        mx = jnp.max(scm)
        mn = jnp.min(scp)
        idx_max = plsc.all_reduce_ffs(scm == mx)
        idx_min = plsc.all_reduce_ffs(scp == mn)
        sel = jnp.where(mx < _THR, idx_min, idx_max)
        neg = jnp.float32(-3.0e38)
        l0s = jnp.max(jnp.where(lane == sel, l0, neg))
        l1s = jnp.max(jnp.where(lane == sel, l1, neg))
        outv[...] = jnp.where(lane == 0, l0s,
                              jnp.where(lane == 1, l1s, 0.0))
        pltpu.sync_copy(outv, feats_hbm.at[pl.ds(bag * _L, _L)])
        outv[...] = jnp.where(valid, sc, 0.0)
        pltpu.sync_copy(outv, scores_hbm.at[pl.ds(bag * _L, _L)])


def kernel(inst_feat, cluster_labels, W_head, b_head):
    B, N, D = inst_feat.shape
    ncls = W_head.shape[1]
    nch = N // _NK

    seg = pl.pallas_call(
        _tc_body,
        grid=(B, _NK),
        in_specs=[
            pl.BlockSpec((D, ncls), lambda b, k: (0, 0)),
            pl.BlockSpec((1, ncls), lambda b, k: (0, 0)),
            pl.BlockSpec(memory_space=pl.ANY),
            pl.BlockSpec((1, nch, 1), lambda b, k: (b, k, 0)),
        ],
        out_specs=pl.BlockSpec((1, _C, 128), lambda b, k: (b, 0, 0)),
        out_shape=jax.ShapeDtypeStruct((B, _C, 128), jnp.float32),
        scratch_shapes=[
            pltpu.VMEM((_NB, nch, D), jnp.float32),
            pltpu.VMEM((_C, D), jnp.float32),
            pltpu.VMEM((_C, _C), jnp.float32),
            pltpu.SemaphoreType.DMA((_NB,)),
        ],
    )(W_head, b_head.reshape(1, ncls), inst_feat,
      cluster_labels.reshape(B, N, 1))

    mesh = plsc.VectorSubcoreMesh(core_axis_name="c", subcore_axis_name="s")
    sc_call = pl.kernel(
        _sc_body,
        out_type=(
            jax.ShapeDtypeStruct((B * _L,), jnp.float32),
            jax.ShapeDtypeStruct((B * _L,), jnp.float32),
        ),
        mesh=mesh,
        compiler_params=pltpu.CompilerParams(needs_layout_passes=False),
        scratch_types=[
            pltpu.VMEM((_C * 128,), jnp.float32),
            pltpu.VMEM((_L,), jnp.float32),
        ],
    )
    featsp, scoresp = sc_call(seg.reshape(-1))
    feats = featsp.reshape(B, _L)[:, :ncls]
    scores = scoresp.reshape(B, _L)[:, :_C]
    return feats, scores
